# Initial kernel scaffold; baseline (speedup 1.0000x reference)
import functools

import jax
import jax.numpy as jnp
from jax import lax
from jax.experimental import pallas as pl
from jax.experimental.pallas import tpu as pltpu
from jax.experimental.pallas import tpu_sc as plsc

DELTA = 2.772588722239781

NT = 4
DT = 64
NW = 32
NB = 320
N_PAD = NW * NB
CS = 2048
FQ = 256
C = 256
MB = 512
BIG = 3.0e38


def _tc1_body(h_ref, wt_ref, wb_ref, bp_ref, a_ref, b_ref):
    hblk = h_ref[...]
    a_ref[0] = jnp.dot(hblk, wt_ref[0], preferred_element_type=jnp.float32)
    b_ref[0] = (jnp.dot(hblk, wb_ref[0], preferred_element_type=jnp.float32)
                + bp_ref[0])


def _tc1(hp, wtop, wbot, b_pre):
    nm = N_PAD // MB
    return pl.pallas_call(
        _tc1_body,
        grid=(nm, NT),
        in_specs=[
            pl.BlockSpec((MB, DT), lambda m, t: (m, t)),
            pl.BlockSpec((1, DT, DT), lambda m, t: (t, 0, 0)),
            pl.BlockSpec((1, DT, DT), lambda m, t: (t, 0, 0)),
            pl.BlockSpec((1, DT), lambda m, t: (t, 0)),
        ],
        out_specs=[
            pl.BlockSpec((1, MB, DT), lambda m, t: (t, m, 0)),
            pl.BlockSpec((1, MB, DT), lambda m, t: (t, m, 0)),
        ],
        out_shape=[
            jax.ShapeDtypeStruct((NT, N_PAD, DT), jnp.float32),
            jax.ShapeDtypeStruct((NT, N_PAD, DT), jnp.float32),
        ],
    )(hp, wtop, wbot, b_pre)


def _make_sc(n_chunks, sc_len):
    mesh = plsc.VectorSubcoreMesh(core_axis_name="c", subcore_axis_name="s")

    @functools.partial(
        pl.kernel,
        mesh=mesh,
        out_type=[
            jax.ShapeDtypeStruct((NT, N_PAD, DT), jnp.float32),
            jax.ShapeDtypeStruct((NT, N_PAD, DT), jnp.float32),
            jax.ShapeDtypeStruct((NT, N_PAD, DT), jnp.float32),
            jax.ShapeDtypeStruct((NT, N_PAD, DT), jnp.float32),
            jax.ShapeDtypeStruct((N_PAD,), jnp.float32),
            jax.ShapeDtypeStruct((NW, sc_len), jnp.int32),
            jax.ShapeDtypeStruct((NW, sc_len), jnp.int32),
        ],
        scratch_types=[
            pltpu.VMEM((CS,), jnp.int32),
            pltpu.VMEM((CS,), jnp.int32),
            pltpu.VMEM((FQ + 16,), jnp.int32),
            pltpu.VMEM((FQ + 16,), jnp.int32),
            pltpu.VMEM((C,), jnp.int32),
            pltpu.VMEM((C,), jnp.int32),
            pltpu.VMEM((C,), jnp.int32),
            pltpu.VMEM((C, DT), jnp.float32),
            pltpu.VMEM((NB, DT), jnp.float32),
            pltpu.VMEM((NB, DT), jnp.float32),
            pltpu.VMEM((NB, DT), jnp.float32),
            pltpu.VMEM((NB, DT), jnp.float32),
            pltpu.VMEM((NB, DT), jnp.float32),
            pltpu.VMEM((NB,), jnp.float32),
            pltpu.SemaphoreType.DMA,
        ],
    )
    def sc_kernel(src_h, dst_h, a4, b4,
                  o_s, o_q, o_x, o_n, o_d, scr_s, scr_d,
                  dstg, srcg, cbuf_s, cbuf_d, sbuf, dlbuf, ibuf, gbuf,
                  bblk, acc_s, acc_q, acc_x, acc_n, dega, sem):
        w = lax.axis_index("s") * 2 + lax.axis_index("c")
        lo = w * NB
        hi = lo + NB

        def chunk_body(ch, carry):
            pltpu.sync_copy(dst_h.at[pl.ds(ch * CS, CS)], dstg)
            pltpu.sync_copy(src_h.at[pl.ds(ch * CS, CS)], srcg)

            def vec_body(i, fw):
                f, woff = fw
                dv = dstg[pl.ds(i * 16, 16)]
                sv = srcg[pl.ds(i * 16, 16)]
                m = (dv >= lo) & (dv < hi)
                plsc.store_compressed(cbuf_d.at[pl.ds(f, 16)], dv - lo, m)
                plsc.store_compressed(cbuf_s.at[pl.ds(f, 16)], sv, m)
                f = f + plsc.all_reduce_population_count(m)[0]

                def flush(fw2):
                    f2, wo2 = fw2
                    pltpu.sync_copy(cbuf_s.at[pl.ds(0, FQ)],
                                    scr_s.at[w, pl.ds(wo2, FQ)])
                    pltpu.sync_copy(cbuf_d.at[pl.ds(0, FQ)],
                                    scr_d.at[w, pl.ds(wo2, FQ)])
                    cbuf_s[pl.ds(0, 16)] = cbuf_s[pl.ds(FQ, 16)]
                    cbuf_d[pl.ds(0, 16)] = cbuf_d[pl.ds(FQ, 16)]
                    return f2 - FQ, wo2 + FQ

                return lax.cond(f >= FQ, flush, lambda fw2: fw2, (f, woff))

            return lax.fori_loop(0, CS // 16, vec_body, carry)

        f, woff = lax.fori_loop(0, n_chunks, chunk_body,
                                (jnp.int32(0), jnp.int32(0)))
        pltpu.sync_copy(cbuf_s.at[pl.ds(0, FQ)], scr_s.at[w, pl.ds(woff, FQ)])
        pltpu.sync_copy(cbuf_d.at[pl.ds(0, FQ)], scr_d.at[w, pl.ds(woff, FQ)])
        cnt = woff + f

        for p in range(NT):
            zero = jnp.zeros((16,), jnp.float32)
            big = jnp.full((16,), BIG, jnp.float32)

            def init_body(i, _, p=p):
                for k in range(DT // 16):
                    acc_s[i, pl.ds(k * 16, 16)] = zero
                    acc_q[i, pl.ds(k * 16, 16)] = zero
                    acc_x[i, pl.ds(k * 16, 16)] = zero
                    acc_n[i, pl.ds(k * 16, 16)] = big
                if p == 0:
                    dega[i] = 0.0
                return 0

            lax.fori_loop(0, NB, init_body, 0)

            pltpu.sync_copy(b4.at[pl.ds(p * N_PAD + lo, NB)], bblk)

            n_gc = (cnt + C - 1) // C

            def gchunk_body(g, _, p=p):
                base = g * C
                pltpu.sync_copy(scr_s.at[w, pl.ds(base, C)], sbuf)
                pltpu.sync_copy(scr_d.at[w, pl.ds(base, C)], dlbuf)

                def adj_body(i, _):
                    v = sbuf[pl.ds(i * 16, 16)]
                    v = jnp.clip(v, 0, N_PAD - 1) + p * N_PAD
                    ibuf[pl.ds(i * 16, 16)] = v
                    return 0

                lax.fori_loop(0, C // 16, adj_body, 0)
                cp0 = pltpu.async_copy(a4.at[ibuf.at[pl.ds(0, 128)]],
                                       gbuf.at[pl.ds(0, 128)], sem)
                cp1 = pltpu.async_copy(a4.at[ibuf.at[pl.ds(128, 128)]],
                                       gbuf.at[pl.ds(128, 128)], sem)
                cp0.wait()
                cp1.wait()

                nc = jnp.minimum(C, cnt - base)

                def edge_body(j, _, p=p):
                    d = dlbuf[j]
                    for k in range(DT // 16):
                        sl = pl.ds(k * 16, 16)
                        msg = jnp.maximum(gbuf[j, sl] + bblk[d, sl], 0.0)
                        plsc.addupdate(acc_s.at[d, sl], msg)
                        plsc.addupdate(acc_q.at[d, sl], msg * msg)
                        acc_x[d, sl] = jnp.maximum(acc_x[d, sl], msg)
                        acc_n[d, sl] = jnp.minimum(acc_n[d, sl], msg)
                    if p == 0:
                        dega[d] = dega[d] + 1.0
                    return 0

                lax.fori_loop(0, nc, edge_body, 0)
                return 0

            lax.fori_loop(0, n_gc, gchunk_body, 0)

            pltpu.sync_copy(acc_s, o_s.at[p, pl.ds(lo, NB)])
            pltpu.sync_copy(acc_q, o_q.at[p, pl.ds(lo, NB)])
            pltpu.sync_copy(acc_x, o_x.at[p, pl.ds(lo, NB)])
            pltpu.sync_copy(acc_n, o_n.at[p, pl.ds(lo, NB)])
            if p == 0:
                pltpu.sync_copy(dega, o_d.at[pl.ds(lo, NB)])

    return sc_kernel


def _tc2_body(h_ref, s_ref, q_ref, x_ref, n_ref, dg_ref, wp_ref, bp_ref,
              wm_ref, bm_ref, out_ref):
    deg = dg_ref[:, 0:1]
    denom = jnp.maximum(deg, 1.0)
    has = deg > 0.0
    logd = jnp.log(deg + 1.0)
    amp = logd * (1.0 / DELTA)
    att = jnp.where(has, DELTA / jnp.maximum(logd, 1e-6), 0.0)
    hblk = h_ref[...]
    outs = []
    for p in range(NT):
        mean = s_ref[p] / denom
        sqm = q_ref[p] / denom
        var = jnp.maximum(sqm - mean * mean, 0.0)
        std = jnp.sqrt(var + 1e-5)
        mxp = jnp.where(has, x_ref[p], 0.0)
        mnp = jnp.where(has, n_ref[p], 0.0)
        xp = jnp.concatenate([mean, mxp, mnp, std], axis=1)
        pre = (jnp.dot(hblk[:, p * DT:(p + 1) * DT], wp_ref[p, 0:DT, :],
                       preferred_element_type=jnp.float32)
               + jnp.dot(xp, wp_ref[p, DT:DT + 256, :],
                         preferred_element_type=jnp.float32)
               + amp * jnp.dot(xp, wp_ref[p, DT + 256:DT + 512, :],
                               preferred_element_type=jnp.float32)
               + att * jnp.dot(xp, wp_ref[p, DT + 512:DT + 768, :],
                               preferred_element_type=jnp.float32)
               + bp_ref[p])
        outs.append(jnp.maximum(pre, 0.0))
    tcat = jnp.concatenate(outs, axis=1)
    mix = jnp.dot(tcat, wm_ref[...], preferred_element_type=jnp.float32)
    out_ref[...] = hblk + jnp.maximum(mix + bm_ref[0], 0.0)


def _tc2(hp, s4, q4, x4, n4, degb, w_post, b_post, w_mix, b_mix2):
    nm = N_PAD // MB
    return pl.pallas_call(
        _tc2_body,
        grid=(nm,),
        in_specs=[
            pl.BlockSpec((MB, NT * DT), lambda m: (m, 0)),
            pl.BlockSpec((NT, MB, DT), lambda m: (0, m, 0)),
            pl.BlockSpec((NT, MB, DT), lambda m: (0, m, 0)),
            pl.BlockSpec((NT, MB, DT), lambda m: (0, m, 0)),
            pl.BlockSpec((NT, MB, DT), lambda m: (0, m, 0)),
            pl.BlockSpec((MB, 128), lambda m: (m, 0)),
            pl.BlockSpec((NT, 832, DT), lambda m: (0, 0, 0)),
            pl.BlockSpec((NT, DT), lambda m: (0, 0)),
            pl.BlockSpec((NT * DT, NT * DT), lambda m: (0, 0)),
            pl.BlockSpec((1, NT * DT), lambda m: (0, 0)),
        ],
        out_specs=pl.BlockSpec((MB, NT * DT), lambda m: (m, 0)),
        out_shape=jax.ShapeDtypeStruct((N_PAD, NT * DT), jnp.float32),
    )(hp, s4, q4, x4, n4, degb, w_post, b_post, w_mix, b_mix2)


def kernel(h, edge_index, e, W_pre, b_pre, W_post, b_post, W_mix, b_mix):
    n, _ = h.shape
    n_edges = edge_index.shape[1]
    src = edge_index[0].astype(jnp.int32)
    dst = edge_index[1].astype(jnp.int32)

    n_chunks = -(-n_edges // CS)
    e_pad = n_chunks * CS
    sc_len = e_pad + FQ + C
    src_p = jnp.pad(src, (0, e_pad - n_edges))
    dst_p = jnp.pad(dst, (0, e_pad - n_edges), constant_values=2 ** 30)

    hp = jnp.pad(h, ((0, N_PAD - n), (0, 0)))
    wtop = W_pre[:, :DT, :]
    wbot = W_pre[:, DT:, :]

    a4, b4 = _tc1(hp, wtop, wbot, b_pre)
    a4f = a4.reshape(NT * N_PAD, DT)
    b4f = b4.reshape(NT * N_PAD, DT)

    sc = _make_sc(n_chunks, sc_len)
    s4, q4, x4, n4, deg, _, _ = sc(src_p, dst_p, a4f, b4f)

    degb = jnp.broadcast_to(deg[:, None], (N_PAD, 128))
    out = _tc2(hp, s4, q4, x4, n4, degb, W_post, b_post, W_mix,
               b_mix.reshape(1, NT * DT))
    return out[:n]


# speedup vs baseline: 4.6257x; 4.6257x over previous
import functools

import jax
import jax.numpy as jnp
from jax import lax
from jax.experimental import pallas as pl
from jax.experimental.pallas import tpu as pltpu
from jax.experimental.pallas import tpu_sc as plsc

DELTA = 2.772588722239781

NT = 4
DT = 64
NW = 32
NB = 320
N_PAD = NW * NB
CS = 2048
FQ = 256
C = 256
MB = 512
BIG = 3.0e38


def _tc1_body(h_ref, wt_ref, wb_ref, bp_ref, a_ref, b_ref):
    hblk = h_ref[...]
    for t in range(NT):
        ht = hblk[:, t * DT:(t + 1) * DT]
        a_ref[t] = jnp.dot(ht, wt_ref[t], preferred_element_type=jnp.float32)
        b_ref[t] = (jnp.dot(ht, wb_ref[t], preferred_element_type=jnp.float32)
                    + bp_ref[t])


def _tc1(hp, wtop, wbot, b_pre):
    nm = N_PAD // MB
    return pl.pallas_call(
        _tc1_body,
        grid=(nm,),
        in_specs=[
            pl.BlockSpec((MB, NT * DT), lambda m: (m, 0)),
            pl.BlockSpec((NT, DT, DT), lambda m: (0, 0, 0)),
            pl.BlockSpec((NT, DT, DT), lambda m: (0, 0, 0)),
            pl.BlockSpec((NT, DT), lambda m: (0, 0)),
        ],
        out_specs=[
            pl.BlockSpec((NT, MB, DT), lambda m: (0, m, 0)),
            pl.BlockSpec((NT, MB, DT), lambda m: (0, m, 0)),
        ],
        out_shape=[
            jax.ShapeDtypeStruct((NT, N_PAD, DT), jnp.float32),
            jax.ShapeDtypeStruct((NT, N_PAD, DT), jnp.float32),
        ],
    )(hp, wtop, wbot, b_pre)


def _make_sc(n_chunks, sc_len):
    mesh = plsc.VectorSubcoreMesh(core_axis_name="c", subcore_axis_name="s")

    @functools.partial(
        pl.kernel,
        mesh=mesh,
        compiler_params=pltpu.CompilerParams(needs_layout_passes=False, use_tc_tiling_on_sc=False),
        out_type=[
            jax.ShapeDtypeStruct((NT, N_PAD, DT), jnp.float32),
            jax.ShapeDtypeStruct((NT, N_PAD, DT), jnp.float32),
            jax.ShapeDtypeStruct((NT, N_PAD, DT), jnp.float32),
            jax.ShapeDtypeStruct((NT, N_PAD, DT), jnp.float32),
            jax.ShapeDtypeStruct((N_PAD,), jnp.float32),
            jax.ShapeDtypeStruct((NW * sc_len,), jnp.int32),
            jax.ShapeDtypeStruct((NW * sc_len,), jnp.int32),
        ],
        scratch_types=[
            pltpu.VMEM((CS,), jnp.int32),
            pltpu.VMEM((CS,), jnp.int32),
            pltpu.VMEM((FQ + 16,), jnp.int32),
            pltpu.VMEM((FQ + 16,), jnp.int32),
            pltpu.VMEM((16,), jnp.int32),
            pltpu.VMEM((16,), jnp.int32),
            pltpu.VMEM((C,), jnp.int32),
            pltpu.VMEM((C + 16,), jnp.int32),
            pltpu.VMEM((C,), jnp.int32),
            pltpu.VMEM((C, DT), jnp.float32),
            pltpu.VMEM((NB, DT), jnp.float32),
            pltpu.VMEM((NB, DT), jnp.float32),
            pltpu.VMEM((NB, DT), jnp.float32),
            pltpu.VMEM((NB, DT), jnp.float32),
            pltpu.VMEM((NB, DT), jnp.float32),
            pltpu.VMEM((NB,), jnp.float32),
            pltpu.SemaphoreType.DMA,
        ],
    )
    def sc_kernel(src_h, dst_h, a4, b4,
                  o_s, o_q, o_x, o_n, o_d, scr_s, scr_d,
                  dstg, srcg, cbuf_s, cbuf_d, tmp_s, tmp_d, sbuf, dlbuf, ibuf, gbuf,
                  bblk, acc_s, acc_q, acc_x, acc_n, dega, sem):
        w = lax.axis_index("s") * 2 + lax.axis_index("c")
        lo = w * NB
        hi = lo + NB
        sbase = w * sc_len

        def chunk_body(ch, carry):
            pltpu.sync_copy(dst_h.at[pl.ds(pl.multiple_of(ch * CS, CS), CS)], dstg)
            pltpu.sync_copy(src_h.at[pl.ds(pl.multiple_of(ch * CS, CS), CS)], srcg)

            def vec_body(i, fw):
                f, woff = fw
                dv = dstg[pl.ds(i * 16, 16)]
                sv = srcg[pl.ds(i * 16, 16)]
                m = (dv >= lo) & (dv < hi)
                plsc.store_compressed(tmp_d.at[pl.ds(0, 16)], dv - lo, mask=m)
                plsc.store_compressed(tmp_s.at[pl.ds(0, 16)], sv, mask=m)
                cbuf_d[pl.ds(f, 16)] = tmp_d[...]
                cbuf_s[pl.ds(f, 16)] = tmp_s[...]
                f = f + plsc.all_reduce_population_count(m)[0]

                def flush(fw2):
                    f2, wo2 = fw2
                    pltpu.sync_copy(cbuf_s.at[pl.ds(0, FQ)],
                                    scr_s.at[pl.ds(pl.multiple_of(sbase + wo2, FQ), FQ)])
                    pltpu.sync_copy(cbuf_d.at[pl.ds(0, FQ)],
                                    scr_d.at[pl.ds(pl.multiple_of(sbase + wo2, FQ), FQ)])
                    cbuf_s[pl.ds(0, 16)] = cbuf_s[pl.ds(FQ, 16)]
                    cbuf_d[pl.ds(0, 16)] = cbuf_d[pl.ds(FQ, 16)]
                    return f2 - FQ, wo2 + FQ

                return lax.cond(f >= FQ, flush, lambda fw2: fw2, (f, woff))

            return lax.fori_loop(0, CS // 16, vec_body, carry)

        f, woff = lax.fori_loop(0, n_chunks, chunk_body,
                                (jnp.int32(0), jnp.int32(0)))
        pltpu.sync_copy(cbuf_s.at[pl.ds(0, FQ)], scr_s.at[pl.ds(pl.multiple_of(sbase + woff, FQ), FQ)])
        pltpu.sync_copy(cbuf_d.at[pl.ds(0, FQ)], scr_d.at[pl.ds(pl.multiple_of(sbase + woff, FQ), FQ)])
        cnt = woff + f

        for p in range(NT):
            zero = jnp.zeros((16,), jnp.float32)
            big = jnp.full((16,), BIG, jnp.float32)

            def init_body(i, _, p=p):
                for k in range(DT // 16):
                    acc_s[i, pl.ds(k * 16, 16)] = zero
                    acc_q[i, pl.ds(k * 16, 16)] = zero
                    acc_x[i, pl.ds(k * 16, 16)] = zero
                    acc_n[i, pl.ds(k * 16, 16)] = big
                return 0

            lax.fori_loop(0, NB, init_body, 0)
            if p == 0:
                def dinit_body(i, _):
                    dega[pl.ds(i * 16, 16)] = jnp.zeros((16,), jnp.float32)
                    return 0
                lax.fori_loop(0, NB // 16, dinit_body, 0)

            pltpu.sync_copy(b4.at[pl.ds(pl.multiple_of(p * N_PAD + lo, NB), NB)], bblk)

            n_gc = (cnt + C - 1) // C

            def gchunk_body(g, _, p=p):
                base = g * C
                pltpu.sync_copy(scr_s.at[pl.ds(pl.multiple_of(sbase + base, C), C)], sbuf)
                pltpu.sync_copy(scr_d.at[pl.ds(pl.multiple_of(sbase + base, C), C)], dlbuf.at[pl.ds(0, C)])

                def adj_body(i, _):
                    v = sbuf[pl.ds(i * 16, 16)]
                    v = jnp.clip(v, 0, N_PAD - 1) + p * N_PAD
                    ibuf[pl.ds(i * 16, 16)] = v
                    return 0

                lax.fori_loop(0, C // 16, adj_body, 0)
                nc = jnp.minimum(C, cnt - base)
                if p == 0:
                    ones = jnp.ones((16,), jnp.float32)

                    def deg_body(i, _):
                        dlv = dlbuf[pl.ds(i * 16, 16)]
                        valid = (lax.iota(jnp.int32, 16) + i * 16) < nc
                        plsc.addupdate_scatter(dega, [dlv], ones, mask=valid)
                        return 0

                    lax.fori_loop(0, C // 16, deg_body, 0)
                cp0 = pltpu.async_copy(a4.at[ibuf.at[pl.ds(0, 128)]],
                                       gbuf.at[pl.ds(0, 128)], sem)
                cp1 = pltpu.async_copy(a4.at[ibuf.at[pl.ds(128, 128)]],
                                       gbuf.at[pl.ds(128, 128)], sem)
                cp0.wait()
                cp1.wait()

                def edge_body(j, _, p=p):
                    d = dlbuf[pl.ds(j, 16)][0]
                    for k in range(DT // 16):
                        sl = pl.ds(k * 16, 16)
                        msg = jnp.maximum(gbuf[j, sl] + bblk[d, sl], 0.0)
                        plsc.addupdate(acc_s.at[d, sl], msg)
                        plsc.addupdate(acc_q.at[d, sl], msg * msg)
                        acc_x[d, sl] = jnp.maximum(acc_x[d, sl], msg)
                        acc_n[d, sl] = jnp.minimum(acc_n[d, sl], msg)
                    return 0

                lax.fori_loop(0, nc, edge_body, 0)
                return 0

            lax.fori_loop(0, n_gc, gchunk_body, 0)

            pltpu.sync_copy(acc_s, o_s.at[p, pl.ds(pl.multiple_of(lo, NB), NB)])
            pltpu.sync_copy(acc_q, o_q.at[p, pl.ds(pl.multiple_of(lo, NB), NB)])
            pltpu.sync_copy(acc_x, o_x.at[p, pl.ds(pl.multiple_of(lo, NB), NB)])
            pltpu.sync_copy(acc_n, o_n.at[p, pl.ds(pl.multiple_of(lo, NB), NB)])
            if p == 0:
                pltpu.sync_copy(dega, o_d.at[pl.ds(pl.multiple_of(lo, NB), NB)])

    return sc_kernel


def _tc2_body(h_ref, s_ref, q_ref, x_ref, n_ref, dg_ref, wp_ref, bp_ref,
              wm_ref, bm_ref, out_ref):
    deg = dg_ref[:, 0:1]
    denom = jnp.maximum(deg, 1.0)
    has = deg > 0.0
    logd = jnp.log(deg + 1.0)
    amp = logd * (1.0 / DELTA)
    att = jnp.where(has, DELTA / jnp.maximum(logd, 1e-6), 0.0)
    hblk = h_ref[...]
    outs = []
    for p in range(NT):
        mean = s_ref[p] / denom
        sqm = q_ref[p] / denom
        var = jnp.maximum(sqm - mean * mean, 0.0)
        std = jnp.sqrt(var + 1e-5)
        mxp = jnp.where(has, x_ref[p], 0.0)
        mnp = jnp.where(has, n_ref[p], 0.0)
        xp = jnp.concatenate([mean, mxp, mnp, std], axis=1)
        pre = (jnp.dot(hblk[:, p * DT:(p + 1) * DT], wp_ref[p, 0:DT, :],
                       preferred_element_type=jnp.float32)
               + jnp.dot(xp, wp_ref[p, DT:DT + 256, :],
                         preferred_element_type=jnp.float32)
               + amp * jnp.dot(xp, wp_ref[p, DT + 256:DT + 512, :],
                               preferred_element_type=jnp.float32)
               + att * jnp.dot(xp, wp_ref[p, DT + 512:DT + 768, :],
                               preferred_element_type=jnp.float32)
               + bp_ref[p])
        outs.append(jnp.maximum(pre, 0.0))
    tcat = jnp.concatenate(outs, axis=1)
    mix = jnp.dot(tcat, wm_ref[...], preferred_element_type=jnp.float32)
    out_ref[...] = hblk + jnp.maximum(mix + bm_ref[0], 0.0)


def _tc2(hp, s4, q4, x4, n4, degb, w_post, b_post, w_mix, b_mix2):
    nm = N_PAD // MB
    return pl.pallas_call(
        _tc2_body,
        grid=(nm,),
        in_specs=[
            pl.BlockSpec((MB, NT * DT), lambda m: (m, 0)),
            pl.BlockSpec((NT, MB, DT), lambda m: (0, m, 0)),
            pl.BlockSpec((NT, MB, DT), lambda m: (0, m, 0)),
            pl.BlockSpec((NT, MB, DT), lambda m: (0, m, 0)),
            pl.BlockSpec((NT, MB, DT), lambda m: (0, m, 0)),
            pl.BlockSpec((MB, 128), lambda m: (m, 0)),
            pl.BlockSpec((NT, 832, DT), lambda m: (0, 0, 0)),
            pl.BlockSpec((NT, DT), lambda m: (0, 0)),
            pl.BlockSpec((NT * DT, NT * DT), lambda m: (0, 0)),
            pl.BlockSpec((1, NT * DT), lambda m: (0, 0)),
        ],
        out_specs=pl.BlockSpec((MB, NT * DT), lambda m: (m, 0)),
        out_shape=jax.ShapeDtypeStruct((N_PAD, NT * DT), jnp.float32),
    )(hp, s4, q4, x4, n4, degb, w_post, b_post, w_mix, b_mix2)


def kernel(h, edge_index, e, W_pre, b_pre, W_post, b_post, W_mix, b_mix):
    n, _ = h.shape
    n_edges = edge_index.shape[1]
    src = edge_index[0].astype(jnp.int32)
    dst = edge_index[1].astype(jnp.int32)

    n_chunks = -(-n_edges // CS)
    e_pad = n_chunks * CS
    sc_len = e_pad + FQ + C
    src_p = jnp.pad(src, (0, e_pad - n_edges))
    dst_p = jnp.pad(dst, (0, e_pad - n_edges), constant_values=2 ** 30)

    hp = jnp.pad(h, ((0, N_PAD - n), (0, 0)))
    wtop = W_pre[:, :DT, :]
    wbot = W_pre[:, DT:, :]

    a4, b4 = _tc1(hp, wtop, wbot, b_pre)
    a4f = a4.reshape(NT * N_PAD, DT)
    b4f = b4.reshape(NT * N_PAD, DT)

    sc = _make_sc(n_chunks, sc_len)
    s4, q4, x4, n4, deg, _, _ = sc(src_p, dst_p, a4f, b4f)

    degb = jnp.broadcast_to(deg[:, None], (N_PAD, 128))
    out = _tc2(hp, s4, q4, x4, n4, degb, W_post, b_post, W_mix,
               b_mix.reshape(1, NT * DT))
    return out[:n]
